# baseline (speedup 1.0000x reference)
import dataclasses
import functools

import jax
import jax.numpy as jnp
from jax import lax
from jax.experimental import pallas as pl
from jax.experimental.pallas import tpu as pltpu
from jax.experimental.pallas import tpu_sc as plsc

N = 10000
E = 320000
H = 128
G = 64

NC = 2
NS = 16
NW = NC * NS
EPT = E // NW
CH = 80
NCHUNK = EPT // CH
RPT = 624
ZR = 104
RTAIL = N - RPT * NS


def _edge_sc(width, kv):
    mesh = plsc.VectorSubcoreMesh(core_axis_name="c", subcore_axis_name="s")
    cp = pltpu.CompilerParams()
    if "needs_layout_passes" in pltpu.CompilerParams.__dataclass_fields__:
        cp = dataclasses.replace(cp, needs_layout_passes=False)

    @functools.partial(
        pl.kernel,
        out_type=jax.ShapeDtypeStruct((NC, N, width), jnp.float32),
        mesh=mesh,
        compiler_params=cp,
        scratch_types=[
            pltpu.VMEM((EPT,), jnp.int32),
            pltpu.VMEM((3, width), jnp.float32),
            pltpu.VMEM((1, width), jnp.float32),
            pltpu.VMEM((CH, width), jnp.float32),
            pltpu.VMEM((CH, width), jnp.float32),
            pltpu.VMEM((CH, width), jnp.float32),
            pltpu.VMEM((3, CH), jnp.float32),
            pltpu.VMEM((3, CH), jnp.float32),
            pltpu.VMEM((3, CH), jnp.float32),
            pltpu.VMEM((1, CH), jnp.int32),
            pltpu.VMEM((1, CH), jnp.int32),
            pltpu.VMEM((1, CH), jnp.int32),
            pltpu.VMEM_SHARED((N, width), jnp.float32),
            pltpu.SemaphoreType.DMA,
            pltpu.SemaphoreType.DMA,
            pltpu.SemaphoreType.DMA,
            pltpu.SemaphoreType.DMA,
            pltpu.SemaphoreType.DMA,
            pltpu.SemaphoreType.DMA,
        ],
    )
    def k(h_hbm, src_hbm, dst_hbm, ea_hbm, lw_hbm, lb_hbm, out_hbm,
          srcv, lwv, lbv, rb0, rb1, rb2, eb0, eb1, eb2, db0, db1, db2,
          agg, sg0, sg1, sg2, ss0, ss1, ss2):
        cid = lax.axis_index("c")
        sid = lax.axis_index("s")
        tid = sid * NC + cid
        rows_b = [rb0, rb1, rb2]
        ea_b = [eb0, eb1, eb2]
        dst_b = [db0, db1, db2]
        sg = [sg0, sg1, sg2]
        ss = [ss0, ss1, ss2]

        def issue_gather(b, p):
            pltpu.async_copy(h_hbm.at[srcv.at[pl.ds(p * CH, CH)]],
                             rows_b[b], sg[b])
            pltpu.async_copy(ea_hbm.at[tid, p], ea_b[b], sg[b])
            pltpu.async_copy(dst_hbm.at[tid, p], dst_b[b], sg[b])

        def wait_gather(b, p):
            pltpu.make_async_copy(h_hbm.at[srcv.at[pl.ds(p * CH, CH)]],
                                  rows_b[b], sg[b]).wait()
            pltpu.make_async_copy(ea_hbm.at[tid, p], ea_b[b], sg[b]).wait()
            pltpu.make_async_copy(dst_hbm.at[tid, p], dst_b[b], sg[b]).wait()

        def issue_scatter(b):
            pltpu.async_copy(rows_b[b], agg.at[dst_b[b].at[0]], ss[b],
                             add=True)

        def wait_scatter(b):
            pltpu.make_async_copy(rows_b[b], agg.at[dst_b[b].at[0]],
                                  ss[b]).wait()

        pltpu.sync_copy(src_hbm.at[tid], srcv)
        pltpu.sync_copy(lw_hbm, lwv)
        pltpu.sync_copy(lb_hbm, lbv)
        issue_gather(0, 0)
        issue_gather(1, 1)

        zv = jnp.zeros((16,), jnp.float32)

        @pl.loop(0, CH)
        def _(r):
            for j in range(width // 16):
                rb2[r, pl.ds(j * 16, 16)] = zv

        soff = sid * RPT
        for j in range(RPT // CH):
            pltpu.sync_copy(rb2, agg.at[pl.ds(soff + j * CH, CH)])
        pltpu.sync_copy(rb2.at[pl.ds(0, RPT - (RPT // CH) * CH)],
                        agg.at[pl.ds(soff + (RPT // CH) * CH,
                                     RPT - (RPT // CH) * CH)])

        @pl.when(sid == NS - 1)
        def _():
            pltpu.sync_copy(rb2.at[pl.ds(0, RTAIL)],
                            agg.at[pl.ds(RPT * NS, RTAIL)])

        plsc.subcore_barrier()

        w0 = [lwv[0, pl.ds(j * 16, 16)] for j in range(kv)]
        w1 = [lwv[1, pl.ds(j * 16, 16)] for j in range(kv)]
        w2 = [lwv[2, pl.ds(j * 16, 16)] for j in range(kv)]
        bb = [lbv[0, pl.ds(j * 16, 16)] for j in range(kv)]

        row0 = jnp.zeros((16,), jnp.int32)
        row1 = row0 + 1
        row2 = row0 + 2

        def compute(b):
            rb, eb = rows_b[b], ea_b[b]

            @pl.loop(0, CH, unroll=2)
            def _(i):
                col = jnp.full((16,), i, jnp.int32)
                a0 = plsc.load_gather(eb, [row0, col])
                a1 = plsc.load_gather(eb, [row1, col])
                a2 = plsc.load_gather(eb, [row2, col])
                for j in range(kv):
                    sl = pl.ds(j * 16, 16)
                    ev = bb[j] + a0 * w0[j] + a1 * w1[j] + a2 * w2[j]
                    rb[i, sl] = jnp.maximum(rb[i, sl] + ev, 0.0)

        @pl.loop(0, NCHUNK - 2, step=3)
        def _(g):
            wait_gather(0, g)
            compute(0)

            @pl.when(g > 0)
            def _():
                wait_scatter(2)
            issue_gather(2, g + 2)
            issue_scatter(0)

            wait_gather(1, g + 1)
            compute(1)
            wait_scatter(0)
            issue_gather(0, g + 3)
            issue_scatter(1)

            wait_gather(2, g + 2)
            compute(2)
            wait_scatter(1)
            issue_gather(1, g + 4)
            issue_scatter(2)

        wait_gather(0, NCHUNK - 2)
        compute(0)
        wait_scatter(2)
        issue_scatter(0)

        wait_gather(1, NCHUNK - 1)
        compute(1)
        wait_scatter(0)
        issue_scatter(1)
        wait_scatter(1)

        plsc.subcore_barrier()
        for j in range(RPT // ZR):
            r0 = soff + j * ZR
            pltpu.sync_copy(agg.at[pl.ds(r0, ZR)],
                            out_hbm.at[cid, pl.ds(r0, ZR)])

        @pl.when(sid == NS - 1)
        def _():
            pltpu.sync_copy(agg.at[pl.ds(RPT * NS, RTAIL)],
                            out_hbm.at[cid, pl.ds(RPT * NS, RTAIL)])

    return k


_edge1 = _edge_sc(H, 1)
_edge128 = _edge_sc(H, 8)


def _mlp(z, p):
    t = z @ p['W1'].T + p['b1']
    mu = jnp.mean(t, axis=0)
    var = jnp.var(t, axis=0)
    r = jax.nn.relu((t - mu) / jnp.sqrt(var + 1e-5) * p['g'] + p['be'])
    return r @ p['W2'].T + p['b2']


def kernel(x, edge_index, edge_attr, batch, params):
    src = edge_index[0].reshape(NW, EPT)
    dst = edge_index[1].reshape(NW, NCHUNK, 1, CH)
    ea = edge_attr.T.reshape(3, NW, NCHUNK, CH).transpose(1, 2, 0, 3)

    c1, c2, c3 = params['c1'], params['c2'], params['c3']
    xp = jnp.pad(x, ((0, 0), (0, H - 6)))
    lw1 = jnp.pad(c1['leW'].T, ((0, 0), (0, H - 6)))
    lb1 = jnp.pad(c1['leb'], (0, H - 6)).reshape(1, H)

    p1 = _edge1(xp, src, dst, ea, lw1, lb1)
    h1 = jax.nn.relu(_mlp(x + (p1[0] + p1[1])[:, :6], c1))

    p2 = _edge128(h1, src, dst, ea, c2['leW'].T, c2['leb'].reshape(1, H))
    h2 = jax.nn.relu(_mlp(h1 + p2[0] + p2[1], c2))

    p3 = _edge128(h2, src, dst, ea, c3['leW'].T, c3['leb'].reshape(1, H))
    h3 = jax.nn.relu(_mlp(h2 + p3[0] + p3[1], c3))

    pooled = jax.ops.segment_sum(h3, batch, num_segments=G)
    emb = jax.nn.relu(pooled @ params['fcW'].T + params['fcb'])
    return emb @ params['outW'].T + params['outb']


# speedup vs baseline: 1.1049x; 1.1049x over previous
import dataclasses
import functools

import jax
import jax.numpy as jnp
from jax import lax
from jax.experimental import pallas as pl
from jax.experimental.pallas import tpu as pltpu
from jax.experimental.pallas import tpu_sc as plsc

N = 10000
E = 320000
H = 128
G = 64

NC = 2
NS = 16
NW = NC * NS
EPT = E // NW
CH = 80
NCHUNK = EPT // CH
RPT = 624
ZR = 104
RTAIL = N - RPT * NS


def _edge_sc(width, kv):
    mesh = plsc.VectorSubcoreMesh(core_axis_name="c", subcore_axis_name="s")
    cp = pltpu.CompilerParams()
    if "needs_layout_passes" in pltpu.CompilerParams.__dataclass_fields__:
        cp = dataclasses.replace(cp, needs_layout_passes=False)

    @functools.partial(
        pl.kernel,
        out_type=jax.ShapeDtypeStruct((NC, N, width), jnp.float32),
        mesh=mesh,
        compiler_params=cp,
        scratch_types=[
            pltpu.VMEM((EPT,), jnp.int32),
            pltpu.VMEM((3, width), jnp.float32),
            pltpu.VMEM((1, width), jnp.float32),
            pltpu.VMEM((CH, width), jnp.float32),
            pltpu.VMEM((CH, width), jnp.float32),
            pltpu.VMEM((CH, width), jnp.float32),
            pltpu.VMEM((3, CH), jnp.float32),
            pltpu.VMEM((3, CH), jnp.float32),
            pltpu.VMEM((3, CH), jnp.float32),
            pltpu.VMEM((1, CH), jnp.int32),
            pltpu.VMEM((1, CH), jnp.int32),
            pltpu.VMEM((1, CH), jnp.int32),
            pltpu.VMEM_SHARED((N, width), jnp.float32),
            pltpu.SemaphoreType.DMA,
            pltpu.SemaphoreType.DMA,
            pltpu.SemaphoreType.DMA,
            pltpu.SemaphoreType.DMA,
            pltpu.SemaphoreType.DMA,
            pltpu.SemaphoreType.DMA,
        ],
    )
    def k(h_hbm, src_hbm, dst_hbm, ea_hbm, lw_hbm, lb_hbm, out_hbm,
          srcv, lwv, lbv, rb0, rb1, rb2, eb0, eb1, eb2, db0, db1, db2,
          agg, sg0, sg1, sg2, ss0, ss1, ss2):
        cid = lax.axis_index("c")
        sid = lax.axis_index("s")
        tid = sid * NC + cid
        rows_b = [rb0, rb1, rb2]
        ea_b = [eb0, eb1, eb2]
        dst_b = [db0, db1, db2]
        sg = [sg0, sg1, sg2]
        ss = [ss0, ss1, ss2]

        def issue_gather(b, p):
            pltpu.async_copy(h_hbm.at[srcv.at[pl.ds(p * CH, CH)]],
                             rows_b[b], sg[b])
            pltpu.async_copy(ea_hbm.at[tid, p], ea_b[b], sg[b])
            pltpu.async_copy(dst_hbm.at[tid, p], dst_b[b], sg[b])

        def wait_gather(b, p):
            pltpu.make_async_copy(h_hbm.at[srcv.at[pl.ds(p * CH, CH)]],
                                  rows_b[b], sg[b]).wait()
            pltpu.make_async_copy(ea_hbm.at[tid, p], ea_b[b], sg[b]).wait()
            pltpu.make_async_copy(dst_hbm.at[tid, p], dst_b[b], sg[b]).wait()

        def issue_scatter(b):
            pltpu.async_copy(rows_b[b], agg.at[dst_b[b].at[0]], ss[b],
                             add=True)

        def wait_scatter(b):
            pltpu.make_async_copy(rows_b[b], agg.at[dst_b[b].at[0]],
                                  ss[b]).wait()

        pltpu.sync_copy(src_hbm.at[tid], srcv)
        pltpu.sync_copy(lw_hbm, lwv)
        pltpu.sync_copy(lb_hbm, lbv)
        issue_gather(0, 0)
        issue_gather(1, 1)

        zv = jnp.zeros((16,), jnp.float32)

        @pl.loop(0, CH)
        def _(r):
            for j in range(width // 16):
                rb2[r, pl.ds(j * 16, 16)] = zv

        soff = sid * RPT
        for j in range(RPT // CH):
            pltpu.sync_copy(rb2, agg.at[pl.ds(soff + j * CH, CH)])
        pltpu.sync_copy(rb2.at[pl.ds(0, RPT - (RPT // CH) * CH)],
                        agg.at[pl.ds(soff + (RPT // CH) * CH,
                                     RPT - (RPT // CH) * CH)])

        @pl.when(sid == NS - 1)
        def _():
            pltpu.sync_copy(rb2.at[pl.ds(0, RTAIL)],
                            agg.at[pl.ds(RPT * NS, RTAIL)])

        plsc.subcore_barrier()

        w0 = [lwv[0, pl.ds(j * 16, 16)] for j in range(kv)]
        w1 = [lwv[1, pl.ds(j * 16, 16)] for j in range(kv)]
        w2 = [lwv[2, pl.ds(j * 16, 16)] for j in range(kv)]
        bb = [lbv[0, pl.ds(j * 16, 16)] for j in range(kv)]

        row0 = jnp.zeros((16,), jnp.int32)
        row1 = row0 + 1
        row2 = row0 + 2

        def compute(b):
            rb, eb = rows_b[b], ea_b[b]

            @pl.loop(0, CH, unroll=2)
            def _(i):
                col = jnp.full((16,), i, jnp.int32)
                a0 = plsc.load_gather(eb, [row0, col])
                a1 = plsc.load_gather(eb, [row1, col])
                a2 = plsc.load_gather(eb, [row2, col])
                for j in range(kv):
                    sl = pl.ds(j * 16, 16)
                    ev = bb[j] + a0 * w0[j] + a1 * w1[j] + a2 * w2[j]
                    rb[i, sl] = jnp.maximum(rb[i, sl] + ev, 0.0)

        @pl.loop(0, NCHUNK - 2, step=3)
        def _(g):
            wait_gather(0, g)
            compute(0)

            @pl.when(g > 0)
            def _():
                wait_scatter(2)
            issue_gather(2, g + 2)
            issue_scatter(0)

            wait_gather(1, g + 1)
            compute(1)
            wait_scatter(0)
            issue_gather(0, g + 3)
            issue_scatter(1)

            wait_gather(2, g + 2)
            compute(2)
            wait_scatter(1)
            issue_gather(1, g + 4)
            issue_scatter(2)

        wait_gather(0, NCHUNK - 2)
        compute(0)
        wait_scatter(2)
        issue_scatter(0)

        wait_gather(1, NCHUNK - 1)
        compute(1)
        wait_scatter(0)
        issue_scatter(1)
        wait_scatter(1)

        plsc.subcore_barrier()
        for j in range(RPT // ZR):
            r0 = soff + j * ZR
            pltpu.sync_copy(agg.at[pl.ds(r0, ZR)],
                            out_hbm.at[cid, pl.ds(r0, ZR)])

        @pl.when(sid == NS - 1)
        def _():
            pltpu.sync_copy(agg.at[pl.ds(RPT * NS, RTAIL)],
                            out_hbm.at[cid, pl.ds(RPT * NS, RTAIL)])

    return k


def _dense_body(h_ref, p_ref, w1_ref, b1_ref, g_ref, be_ref, w2_ref, b2_ref,
                out_ref):
    z = h_ref[...] + p_ref[0] + p_ref[1]
    t = lax.dot_general(z, w1_ref[...], (((1,), (1,)), ((), ())),
                        preferred_element_type=jnp.float32) + b1_ref[...]
    mu = jnp.mean(t, axis=0, keepdims=True)
    d = t - mu
    var = jnp.mean(d * d, axis=0, keepdims=True)
    r = jnp.maximum(d * lax.rsqrt(var + 1e-5) * g_ref[...] + be_ref[...], 0.0)
    h2 = lax.dot_general(r, w2_ref[...], (((1,), (1,)), ((), ())),
                         preferred_element_type=jnp.float32) + b2_ref[...]
    out_ref[...] = jnp.maximum(h2, 0.0)


def _dense(h, parts, w1, b1, g, be, w2, b2):
    return pl.pallas_call(
        _dense_body,
        out_shape=jax.ShapeDtypeStruct((N, H), jnp.float32),
    )(h, parts, w1, b1, g, be, w2, b2)


def _final_body(h_ref, p_ref, w1_ref, b1_ref, g_ref, be_ref, w2_ref, b2_ref,
                batch_ref, fcw_ref, fcb_ref, ow_ref, ob_ref, out_ref):
    z = h_ref[...] + p_ref[0] + p_ref[1]
    t = lax.dot_general(z, w1_ref[...], (((1,), (1,)), ((), ())),
                        preferred_element_type=jnp.float32) + b1_ref[...]
    mu = jnp.mean(t, axis=0, keepdims=True)
    d = t - mu
    var = jnp.mean(d * d, axis=0, keepdims=True)
    r = jnp.maximum(d * lax.rsqrt(var + 1e-5) * g_ref[...] + be_ref[...], 0.0)
    h2 = lax.dot_general(r, w2_ref[...], (((1,), (1,)), ((), ())),
                         preferred_element_type=jnp.float32) + b2_ref[...]
    h3 = jnp.maximum(h2, 0.0)
    seg = lax.broadcasted_iota(jnp.int32, (G, N), 0)
    oh = (seg == batch_ref[...]).astype(jnp.float32)
    pooled = lax.dot_general(oh, h3, (((1,), (0,)), ((), ())),
                             preferred_element_type=jnp.float32)
    emb = jnp.maximum(
        lax.dot_general(pooled, fcw_ref[...], (((1,), (1,)), ((), ())),
                        preferred_element_type=jnp.float32) + fcb_ref[...],
        0.0)
    out_ref[...] = lax.dot_general(
        emb, ow_ref[...], (((1,), (1,)), ((), ())),
        preferred_element_type=jnp.float32) + ob_ref[...]


def _final(h, parts, w1, b1, g, be, w2, b2, batch2d, fcw, fcb, ow, ob):
    return pl.pallas_call(
        _final_body,
        out_shape=jax.ShapeDtypeStruct((G, 2), jnp.float32),
    )(h, parts, w1, b1, g, be, w2, b2, batch2d, fcw, fcb, ow, ob)


_edge1 = _edge_sc(H, 1)
_edge128 = _edge_sc(H, 8)


def kernel(x, edge_index, edge_attr, batch, params):
    src = edge_index[0].reshape(NW, EPT)
    dst = edge_index[1].reshape(NW, NCHUNK, 1, CH)
    ea = edge_attr.T.reshape(3, NW, NCHUNK, CH).transpose(1, 2, 0, 3)

    c1, c2, c3 = params['c1'], params['c2'], params['c3']
    xp = jnp.pad(x, ((0, 0), (0, H - 6)))
    lw1 = jnp.pad(c1['leW'].T, ((0, 0), (0, H - 6)))
    lb1 = jnp.pad(c1['leb'], (0, H - 6)).reshape(1, H)
    w1p = jnp.pad(c1['W1'], ((0, 0), (0, H - 6)))

    p1 = _edge1(xp, src, dst, ea, lw1, lb1)
    h1 = _dense(xp, p1, w1p, c1['b1'].reshape(1, H), c1['g'].reshape(1, H),
                c1['be'].reshape(1, H), c1['W2'], c1['b2'].reshape(1, H))

    p2 = _edge128(h1, src, dst, ea, c2['leW'].T, c2['leb'].reshape(1, H))
    h2 = _dense(h1, p2, c2['W1'], c2['b1'].reshape(1, H),
                c2['g'].reshape(1, H), c2['be'].reshape(1, H),
                c2['W2'], c2['b2'].reshape(1, H))

    p3 = _edge128(h2, src, dst, ea, c3['leW'].T, c3['leb'].reshape(1, H))
    out = _final(h2, p3, c3['W1'], c3['b1'].reshape(1, H),
                 c3['g'].reshape(1, H), c3['be'].reshape(1, H),
                 c3['W2'], c3['b2'].reshape(1, H),
                 batch.reshape(1, N),
                 params['fcW'], params['fcb'].reshape(1, 256),
                 params['outW'], params['outb'].reshape(1, 2))
    return out
